# baseline (speedup 1.0000x reference)
import functools

import jax
import jax.numpy as jnp
import numpy as np
from jax.experimental import pallas as pl
from jax.experimental.pallas import tpu as pltpu

B, S, D, H, E, K = 1, 2048, 1024, 16, 8, 2
HID = 4 * D
DH = D // H
HBLK = 1024
NH = HID // HBLK
BF = jnp.bfloat16

_NN = (((1,), (0,)), ((), ()))
_TT = (((0,), (0,)), ((), ()))


def _dot(a, b):
    return jax.lax.dot_general(a.astype(BF), b.astype(BF), _NN,
                               preferred_element_type=jnp.float32)


def _lnT(xT, g, b):
    m = jnp.mean(xT, axis=0, keepdims=True)
    v = jnp.mean((xT - m) ** 2, axis=0, keepdims=True)
    return (xT - m) * jax.lax.rsqrt(v + 1e-5) * g + b


def _ln2_kernel(xs_ref, xc_ref, gs_ref, bs_ref, gc_ref, bc_ref,
                x0_ref, xn_ref):
    g = pl.program_id(0)

    @pl.when(g == 0)
    def _():
        t = xs_ref[...].T
        x0_ref[0] = t
        xn_ref[0] = _lnT(t, gs_ref[...], bs_ref[...]).astype(BF)

    @pl.when(g == 1)
    def _():
        t = xc_ref[...].T
        x0_ref[0] = t
        xn_ref[0] = _lnT(t, gc_ref[...], bc_ref[...]).astype(BF)


def _ln2(xs, xc, gs, bs, gc, bc):
    vec = lambda g: (0, 0)
    return pl.pallas_call(
        _ln2_kernel,
        grid=(2,),
        in_specs=[pl.BlockSpec((S, D), vec), pl.BlockSpec((S, D), vec),
                  pl.BlockSpec((D, 1), vec), pl.BlockSpec((D, 1), vec),
                  pl.BlockSpec((D, 1), vec), pl.BlockSpec((D, 1), vec)],
        out_specs=(pl.BlockSpec((1, D, S), lambda g: (g, 0, 0)),
                   pl.BlockSpec((1, D, S), lambda g: (g, 0, 0))),
        out_shape=(jax.ShapeDtypeStruct((2, D, S), jnp.float32),
                   jax.ShapeDtypeStruct((2, D, S), BF)),
    )(xs, xc, gs.reshape(D, 1), bs.reshape(D, 1), gc.reshape(D, 1),
      bc.reshape(D, 1))


def _stage_kernel(xn_ref, wa_ref, wb_ref, ba_ref, bb_ref, o_ref, qkv_scr):
    g = pl.program_id(0)
    j = pl.program_id(1)

    @pl.when(j < 3)
    def _():
        scale = jnp.where(j == 0, np.float32(1.0 / np.sqrt(DH)),
                          np.float32(1.0))

        @pl.when(g == 0)
        def _():
            out = _dot(wa_ref[...], xn_ref[0]) + ba_ref[0]
            qkv_scr[pl.ds(j * D, D), :] = (out * scale).astype(BF)

        @pl.when(g == 1)
        def _():
            out = _dot(wb_ref[...], xn_ref[0]) + bb_ref[0]
            qkv_scr[pl.ds(j * D, D), :] = (out * scale).astype(BF)

    @pl.when(j >= 3)
    def _():
        h = j - 3
        q = qkv_scr[pl.ds(h * DH, DH), :]
        k = qkv_scr[pl.ds(D + h * DH, DH), :]
        v = qkv_scr[pl.ds(2 * D + h * DH, DH), :]
        sT = jax.lax.dot_general(k, q, _TT,
                                 preferred_element_type=jnp.float32)
        eT = jnp.exp(sT)
        v_aug = jnp.concatenate([v, jnp.ones((8, S), BF)], axis=0)
        oT_aug = _dot(v_aug, eT)
        rs = oT_aug[DH:DH + 1, :]
        o_ref[0] = (oT_aug[:DH] * (1.0 / rs)).astype(BF)


def _stage(xn, in_w_a, in_b_a, in_w_b, in_b_b, cross):
    if cross:
        xn_idx = lambda g, j: (jnp.where(j < 1, g, 1 - g), 0, 0)
    else:
        xn_idx = lambda g, j: (g, 0, 0)
    wa_idx = lambda g, j: (jnp.where(g == 0, jnp.minimum(j, 2), 2), 0)
    wb_idx = lambda g, j: (jnp.where(g == 1, jnp.minimum(j, 2), 0), 0)
    return pl.pallas_call(
        _stage_kernel,
        grid=(2, 3 + H),
        in_specs=[
            pl.BlockSpec((1, D, S), xn_idx),
            pl.BlockSpec((D, D), wa_idx),
            pl.BlockSpec((D, D), wb_idx),
            pl.BlockSpec((1, D, 1),
                         lambda g, j: (jnp.where(g == 0, jnp.minimum(j, 2), 2), 0, 0)),
            pl.BlockSpec((1, D, 1),
                         lambda g, j: (jnp.where(g == 1, jnp.minimum(j, 2), 0), 0, 0)),
        ],
        out_specs=pl.BlockSpec((1, DH, S),
                               lambda g, j: (g, jnp.maximum(j - 3, 0), 0)),
        out_shape=jax.ShapeDtypeStruct((2, D, S), BF),
        scratch_shapes=[pltpu.VMEM((3 * D, S), BF)],
    )(xn, in_w_a, in_w_b, in_b_a.reshape(3, D, 1), in_b_b.reshape(3, D, 1))


SB = 2
SBLK = S // SB


def _proj2_ln_kernel(o_ref, wa_ref, wb_ref, ba_ref, bb_ref, res_ref,
                     ga_ref, bga_ref, gb_ref, bgb_ref, x_ref, tn_ref):
    g = pl.program_id(0)

    @pl.when(g == 0)
    def _():
        out = _dot(wa_ref[...], o_ref[0]) + ba_ref[...] + res_ref[0]
        x_ref[0] = out
        tn_ref[0] = _lnT(out, ga_ref[...], bga_ref[...]).astype(BF)

    @pl.when(g == 1)
    def _():
        out = _dot(wb_ref[...], o_ref[0]) + bb_ref[...] + res_ref[0]
        x_ref[0] = out
        tn_ref[0] = _lnT(out, gb_ref[...], bgb_ref[...]).astype(BF)


def _proj2_ln(o2, wa, ba, wb, bb, res2, ga, bga, gb, bgb):
    vec = lambda g, s: (0, 0)
    return pl.pallas_call(
        _proj2_ln_kernel,
        grid=(2, SB),
        in_specs=[
            pl.BlockSpec((1, D, SBLK), lambda g, s: (g, 0, s)),
            pl.BlockSpec((D, D), vec),
            pl.BlockSpec((D, D), vec),
            pl.BlockSpec((D, 1), vec),
            pl.BlockSpec((D, 1), vec),
            pl.BlockSpec((1, D, SBLK), lambda g, s: (g, 0, s)),
            pl.BlockSpec((D, 1), vec),
            pl.BlockSpec((D, 1), vec),
            pl.BlockSpec((D, 1), vec),
            pl.BlockSpec((D, 1), vec),
        ],
        out_specs=(pl.BlockSpec((1, D, SBLK), lambda g, s: (g, 0, s)),
                   pl.BlockSpec((1, D, SBLK), lambda g, s: (g, 0, s))),
        out_shape=(jax.ShapeDtypeStruct((2, D, S), jnp.float32),
                   jax.ShapeDtypeStruct((2, D, S), BF)),
    )(o2, wa, wb, ba.reshape(D, 1), bb.reshape(D, 1), res2,
      ga.reshape(D, 1), bga.reshape(D, 1), gb.reshape(D, 1), bgb.reshape(D, 1))


def _proj2_moe_kernel(o_ref, wa_ref, wb_ref, ba_ref, bb_ref, res_ref,
                      g_ref, b_ref, rw_ref, rb_ref, eps_ref,
                      x_ref, hn_ref, nm_ref):
    sb = pl.program_id(0)
    g = pl.program_id(1)

    @pl.when(g == 0)
    def _():
        x_ref[...] = _dot(wa_ref[...], o_ref[0]) + ba_ref[...] + res_ref[0]

    @pl.when(g == 1)
    def _():
        x = x_ref[...] + _dot(wb_ref[...], o_ref[0]) + bb_ref[...] + res_ref[0]
        x_ref[...] = x
        hn = _lnT(x, g_ref[...], b_ref[...])
        hn_ref[...] = hn.astype(BF)
        lg = jax.lax.dot_general(rw_ref[...].astype(BF), hn.astype(BF), _NN,
                                 preferred_element_type=jnp.float32) + rb_ref[...]
        sp = jnp.log1p(jnp.exp(-jnp.abs(lg[E:]))) + jnp.maximum(lg[E:], 0.0)
        noisy = lg[:E] + eps_ref[...].T * sp
        part = jnp.sum(noisy, axis=1, keepdims=True) * np.float32(1.0 / S)

        @pl.when(sb == 0)
        def _():
            nm_ref[...] = part

        @pl.when(sb > 0)
        def _():
            nm_ref[...] += part


def _proj2_moe(o2, wa, ba, wb, bb, res2, g, b, rw, rb, eps):
    vec = lambda s, g: (0, 0)
    return pl.pallas_call(
        _proj2_moe_kernel,
        grid=(SB, 2),
        in_specs=[
            pl.BlockSpec((1, D, SBLK), lambda s, g: (g, 0, s)),
            pl.BlockSpec((D, D), vec),
            pl.BlockSpec((D, D), vec),
            pl.BlockSpec((D, 1), vec),
            pl.BlockSpec((D, 1), vec),
            pl.BlockSpec((1, D, SBLK), lambda s, g: (g, 0, s)),
            pl.BlockSpec((D, 1), vec),
            pl.BlockSpec((D, 1), vec),
            pl.BlockSpec((2 * E, D), vec),
            pl.BlockSpec((2 * E, 1), vec),
            pl.BlockSpec((SBLK, E), lambda s, g: (s, 0)),
        ],
        out_specs=(pl.BlockSpec((D, SBLK), lambda s, g: (0, s)),
                   pl.BlockSpec((D, SBLK), lambda s, g: (0, s)),
                   pl.BlockSpec((E, 1), vec)),
        out_shape=(jax.ShapeDtypeStruct((D, S), jnp.float32),
                   jax.ShapeDtypeStruct((D, S), BF),
                   jax.ShapeDtypeStruct((E, 1), jnp.float32)),
    )(o2, wa, wb, ba.reshape(D, 1), bb.reshape(D, 1), res2,
      g.reshape(D, 1), b.reshape(D, 1), rw, rb.reshape(2 * E, 1), eps)


def _expert_kernel(idx_ref, probs_ref, hn_ref, x_ref, fc1_ref, b1_ref,
                   fc2_ref, b2_ref, out_ref):
    ki = pl.program_id(0)
    j = pl.program_id(1)
    p = probs_ref[0, ki]
    h1 = _dot(fc1_ref[0], hn_ref[...]) + b1_ref[0]
    h1 = h1 * 0.5 * (1.0 + jax.lax.erf(h1 * np.float32(1.0 / np.sqrt(2.0))))
    part = _dot(fc2_ref[0], h1)

    @pl.when((ki == 0) & (j == 0))
    def _():
        out_ref[...] = x_ref[...]

    @pl.when(j == 0)
    def _():
        out_ref[...] += p * b2_ref[0]

    out_ref[...] += p * part


def _experts(idx, probs, hnT, xT, fc1, b1, fc2, b2):
    grid = (K, NH)
    return pl.pallas_call(
        _expert_kernel,
        grid_spec=pltpu.PrefetchScalarGridSpec(
            num_scalar_prefetch=1,
            grid=grid,
            in_specs=[
                pl.BlockSpec(memory_space=pltpu.SMEM),
                pl.BlockSpec((D, S), lambda k, j, idx: (0, 0)),
                pl.BlockSpec((D, S), lambda k, j, idx: (0, 0)),
                pl.BlockSpec((1, HBLK, D), lambda k, j, idx: (idx[k], j, 0)),
                pl.BlockSpec((1, HBLK, 1), lambda k, j, idx: (idx[k], j, 0)),
                pl.BlockSpec((1, D, HBLK), lambda k, j, idx: (idx[k], 0, j)),
                pl.BlockSpec((1, D, 1), lambda k, j, idx: (idx[k], 0, 0)),
            ],
            out_specs=pl.BlockSpec((D, S), lambda k, j, idx: (0, 0)),
        ),
        out_shape=jax.ShapeDtypeStruct((D, S), jnp.float32),
    )(idx, probs, hnT, xT, fc1, b1, fc2, b2)


def kernel(x_swin, x_conv, noise_eps, params):
    p = params
    x0, xn = _ln2(x_swin.reshape(S, D), x_conv.reshape(S, D),
                  p['swin_pre_ln_g'], p['swin_pre_ln_b'],
                  p['conv_pre_ln_g'], p['conv_pre_ln_b'])
    oc = _stage(xn, p['cross_swin_in_w'], p['cross_swin_in_b'],
                p['cross_conv_in_w'], p['cross_conv_in_b'], cross=True)
    x2, tn = _proj2_ln(oc, p['cross_swin_out_w'], p['cross_swin_out_b'],
                       p['cross_conv_out_w'], p['cross_conv_out_b'],
                       x0,
                       p['swin_self_ln_g'], p['swin_self_ln_b'],
                       p['conv_self_ln_g'], p['conv_self_ln_b'])
    os_ = _stage(tn, p['self_swin_in_w'], p['self_swin_in_b'],
                 p['self_conv_in_w'], p['self_conv_in_b'], cross=False)
    rw = jnp.concatenate([p['router_w'], p['noise_w']], axis=0)
    rb = jnp.concatenate([p['router_b'], p['noise_b']], axis=0)
    xT, hnT, nm = _proj2_moe(os_, p['self_swin_out_w'], p['self_swin_out_b'],
                             p['self_conv_out_w'], p['self_conv_out_b'],
                             x2, p['moe_ln_g'], p['moe_ln_b'], rw, rb,
                             noise_eps.reshape(S, E))

    nmv = nm.reshape(E)
    iota = jnp.arange(E, dtype=jnp.int32)
    m1 = jnp.max(nmv)
    i1 = jnp.min(jnp.where(nmv == m1, iota, E))
    m2 = jnp.max(jnp.where(iota == i1, -jnp.inf, nmv))
    i2 = jnp.min(jnp.where((nmv == m2) & (iota != i1), iota, E))
    e2 = jnp.exp(m2 - m1)
    probs = (jnp.stack([1.0, e2]) / (1.0 + e2)).reshape(1, K)
    idx = jnp.stack([i1, i2])

    outT = _experts(idx.astype(jnp.int32), probs, hnT, xT,
                    p['exp_fc1_w'], p['exp_fc1_b'].reshape(E, HID, 1),
                    p['exp_fc2_w'], p['exp_fc2_b'].reshape(E, D, 1))
    return outT.T.reshape(B, S, D)


# speedup vs baseline: 1.0063x; 1.0063x over previous
import functools

import jax
import jax.numpy as jnp
import numpy as np
from jax.experimental import pallas as pl
from jax.experimental.pallas import tpu as pltpu

B, S, D, H, E, K = 1, 2048, 1024, 16, 8, 2
HID = 4 * D
DH = D // H
HBLK = 1024
NH = HID // HBLK
BF = jnp.bfloat16

_NN = (((1,), (0,)), ((), ()))
_TT = (((0,), (0,)), ((), ()))


def _dot(a, b):
    return jax.lax.dot_general(a.astype(BF), b.astype(BF), _NN,
                               preferred_element_type=jnp.float32)


def _lnT(xT, g, b):
    m = jnp.mean(xT, axis=0, keepdims=True)
    v = jnp.mean((xT - m) ** 2, axis=0, keepdims=True)
    return (xT - m) * jax.lax.rsqrt(v + 1e-5) * g + b


def _ln2_kernel(xs_ref, xc_ref, gs_ref, bs_ref, gc_ref, bc_ref,
                x0_ref, xn_ref):
    g = pl.program_id(0)

    @pl.when(g == 0)
    def _():
        t = xs_ref[...].T
        x0_ref[0] = t
        xn_ref[0] = _lnT(t, gs_ref[...], bs_ref[...]).astype(BF)

    @pl.when(g == 1)
    def _():
        t = xc_ref[...].T
        x0_ref[0] = t
        xn_ref[0] = _lnT(t, gc_ref[...], bc_ref[...]).astype(BF)


def _ln2(xs, xc, gs, bs, gc, bc):
    vec = lambda g: (0, 0)
    return pl.pallas_call(
        _ln2_kernel,
        grid=(2,),
        in_specs=[pl.BlockSpec((S, D), vec), pl.BlockSpec((S, D), vec),
                  pl.BlockSpec((D, 1), vec), pl.BlockSpec((D, 1), vec),
                  pl.BlockSpec((D, 1), vec), pl.BlockSpec((D, 1), vec)],
        out_specs=(pl.BlockSpec((1, D, S), lambda g: (g, 0, 0)),
                   pl.BlockSpec((1, D, S), lambda g: (g, 0, 0))),
        out_shape=(jax.ShapeDtypeStruct((2, D, S), jnp.float32),
                   jax.ShapeDtypeStruct((2, D, S), BF)),
    )(xs, xc, gs.reshape(D, 1), bs.reshape(D, 1), gc.reshape(D, 1),
      bc.reshape(D, 1))


def _stage_kernel(xn_ref, wa_ref, wb_ref, ba_ref, bb_ref, o_ref, qkv_scr):
    g = pl.program_id(0)
    j = pl.program_id(1)

    @pl.when(j < 3)
    def _():
        scale = jnp.where(j == 0, np.float32(1.0 / np.sqrt(DH)),
                          np.float32(1.0))

        @pl.when(g == 0)
        def _():
            out = _dot(wa_ref[...], xn_ref[0]) + ba_ref[0]
            qkv_scr[pl.ds(j * D, D), :] = (out * scale).astype(BF)

        @pl.when(g == 1)
        def _():
            out = _dot(wb_ref[...], xn_ref[0]) + bb_ref[0]
            qkv_scr[pl.ds(j * D, D), :] = (out * scale).astype(BF)

    @pl.when(j >= 3)
    def _():
        h = j - 3
        q = qkv_scr[pl.ds(h * DH, DH), :]
        k = qkv_scr[pl.ds(D + h * DH, DH), :]
        v = qkv_scr[pl.ds(2 * D + h * DH, DH), :]
        sT = jax.lax.dot_general(k, q, _TT,
                                 preferred_element_type=jnp.float32)
        eT = jnp.exp(sT)
        v_aug = jnp.concatenate([v, jnp.ones((8, S), BF)], axis=0)
        oT_aug = _dot(v_aug, eT)
        rs = oT_aug[DH:DH + 1, :]
        o_ref[0] = (oT_aug[:DH] * (1.0 / rs)).astype(BF)


def _stage(xn, in_w_a, in_b_a, in_w_b, in_b_b, cross):
    if cross:
        xn_idx = lambda g, j: (jnp.where(j < 1, g, 1 - g), 0, 0)
    else:
        xn_idx = lambda g, j: (g, 0, 0)
    wa_idx = lambda g, j: (jnp.where(g == 0, jnp.minimum(j, 2), 2), 0)
    wb_idx = lambda g, j: (jnp.where(g == 1, jnp.minimum(j, 2), 0), 0)
    return pl.pallas_call(
        _stage_kernel,
        grid=(2, 3 + H),
        in_specs=[
            pl.BlockSpec((1, D, S), xn_idx),
            pl.BlockSpec((D, D), wa_idx),
            pl.BlockSpec((D, D), wb_idx),
            pl.BlockSpec((1, D, 1),
                         lambda g, j: (jnp.where(g == 0, jnp.minimum(j, 2), 2), 0, 0)),
            pl.BlockSpec((1, D, 1),
                         lambda g, j: (jnp.where(g == 1, jnp.minimum(j, 2), 0), 0, 0)),
        ],
        out_specs=pl.BlockSpec((1, DH, S),
                               lambda g, j: (g, jnp.maximum(j - 3, 0), 0)),
        out_shape=jax.ShapeDtypeStruct((2, D, S), BF),
        scratch_shapes=[pltpu.VMEM((3 * D, S), BF)],
    )(xn, in_w_a, in_w_b, in_b_a.reshape(3, D, 1), in_b_b.reshape(3, D, 1))


SB = 2
SBLK = S // SB


def _proj2_ln_kernel(o_ref, wa_ref, wb_ref, ba_ref, bb_ref, res_ref,
                     ga_ref, bga_ref, gb_ref, bgb_ref, x_ref, tn_ref):
    g = pl.program_id(0)

    @pl.when(g == 0)
    def _():
        out = _dot(wa_ref[...], o_ref[0]) + ba_ref[...] + res_ref[0]
        x_ref[0] = out
        tn_ref[0] = _lnT(out, ga_ref[...], bga_ref[...]).astype(BF)

    @pl.when(g == 1)
    def _():
        out = _dot(wb_ref[...], o_ref[0]) + bb_ref[...] + res_ref[0]
        x_ref[0] = out
        tn_ref[0] = _lnT(out, gb_ref[...], bgb_ref[...]).astype(BF)


def _proj2_ln(o2, wa, ba, wb, bb, res2, ga, bga, gb, bgb):
    vec = lambda g, s: (0, 0)
    return pl.pallas_call(
        _proj2_ln_kernel,
        grid=(2, SB),
        in_specs=[
            pl.BlockSpec((1, D, SBLK), lambda g, s: (g, 0, s)),
            pl.BlockSpec((D, D), vec),
            pl.BlockSpec((D, D), vec),
            pl.BlockSpec((D, 1), vec),
            pl.BlockSpec((D, 1), vec),
            pl.BlockSpec((1, D, SBLK), lambda g, s: (g, 0, s)),
            pl.BlockSpec((D, 1), vec),
            pl.BlockSpec((D, 1), vec),
            pl.BlockSpec((D, 1), vec),
            pl.BlockSpec((D, 1), vec),
        ],
        out_specs=(pl.BlockSpec((1, D, SBLK), lambda g, s: (g, 0, s)),
                   pl.BlockSpec((1, D, SBLK), lambda g, s: (g, 0, s))),
        out_shape=(jax.ShapeDtypeStruct((2, D, S), jnp.float32),
                   jax.ShapeDtypeStruct((2, D, S), BF)),
    )(o2, wa, wb, ba.reshape(D, 1), bb.reshape(D, 1), res2,
      ga.reshape(D, 1), bga.reshape(D, 1), gb.reshape(D, 1), bgb.reshape(D, 1))


def _proj2_moe_kernel(o_ref, wa_ref, wb_ref, ba_ref, bb_ref, res_ref,
                      g_ref, b_ref, rw_ref, rb_ref, eps_ref,
                      x_ref, hn_ref, nm_ref):
    sb = pl.program_id(0)
    g = pl.program_id(1)

    @pl.when(g == 0)
    def _():
        x_ref[...] = _dot(wa_ref[...], o_ref[0]) + ba_ref[...] + res_ref[0]

    @pl.when(g == 1)
    def _():
        x = x_ref[...] + _dot(wb_ref[...], o_ref[0]) + bb_ref[...] + res_ref[0]
        x_ref[...] = x
        hn = _lnT(x, g_ref[...], b_ref[...])
        hn_ref[...] = hn.astype(BF)
        lg = jax.lax.dot_general(rw_ref[...].astype(BF), hn.astype(BF), _NN,
                                 preferred_element_type=jnp.float32) + rb_ref[...]
        sp = jnp.log1p(jnp.exp(-jnp.abs(lg[E:]))) + jnp.maximum(lg[E:], 0.0)
        noisy = lg[:E] + eps_ref[...].T * sp
        part = jnp.sum(noisy, axis=1, keepdims=True) * np.float32(1.0 / S)

        @pl.when(sb == 0)
        def _():
            nm_ref[...] = part

        @pl.when(sb > 0)
        def _():
            nm_ref[...] += part


def _proj2_moe(o2, wa, ba, wb, bb, res2, g, b, rw, rb, eps):
    vec = lambda s, g: (0, 0)
    return pl.pallas_call(
        _proj2_moe_kernel,
        grid=(SB, 2),
        in_specs=[
            pl.BlockSpec((1, D, SBLK), lambda s, g: (g, 0, s)),
            pl.BlockSpec((D, D), vec),
            pl.BlockSpec((D, D), vec),
            pl.BlockSpec((D, 1), vec),
            pl.BlockSpec((D, 1), vec),
            pl.BlockSpec((1, D, SBLK), lambda s, g: (g, 0, s)),
            pl.BlockSpec((D, 1), vec),
            pl.BlockSpec((D, 1), vec),
            pl.BlockSpec((2 * E, D), vec),
            pl.BlockSpec((2 * E, 1), vec),
            pl.BlockSpec((SBLK, E), lambda s, g: (s, 0)),
        ],
        out_specs=(pl.BlockSpec((D, SBLK), lambda s, g: (0, s)),
                   pl.BlockSpec((D, SBLK), lambda s, g: (0, s)),
                   pl.BlockSpec((E, 1), vec)),
        out_shape=(jax.ShapeDtypeStruct((D, S), jnp.float32),
                   jax.ShapeDtypeStruct((D, S), BF),
                   jax.ShapeDtypeStruct((E, 1), jnp.float32)),
    )(o2, wa, wb, ba.reshape(D, 1), bb.reshape(D, 1), res2,
      g.reshape(D, 1), b.reshape(D, 1), rw, rb.reshape(2 * E, 1), eps)


def _expert_kernel(idx_ref, probs_ref, hn_ref, x_ref, fc1_ref, b1_ref,
                   fc2_ref, b2_ref, out_ref):
    ki = pl.program_id(0)
    j = pl.program_id(1)
    p = probs_ref[0, ki]
    h1 = _dot(fc1_ref[0], hn_ref[...]) + b1_ref[0]
    h1 = h1 * 0.5 * (1.0 + jax.lax.erf(h1 * np.float32(1.0 / np.sqrt(2.0))))
    part = _dot(fc2_ref[0], h1)

    @pl.when((ki == 0) & (j == 0))
    def _():
        out_ref[...] = x_ref[...]

    @pl.when(j == 0)
    def _():
        out_ref[...] += p * b2_ref[0]

    out_ref[...] += p * part


def _experts(idx, probs, hnT, xT, fc1, b1, fc2, b2):
    grid = (K, NH)
    return pl.pallas_call(
        _expert_kernel,
        grid_spec=pltpu.PrefetchScalarGridSpec(
            num_scalar_prefetch=1,
            grid=grid,
            in_specs=[
                pl.BlockSpec(memory_space=pltpu.SMEM),
                pl.BlockSpec((D, S), lambda k, j, idx: (0, 0)),
                pl.BlockSpec((D, S), lambda k, j, idx: (0, 0)),
                pl.BlockSpec((1, HBLK, D), lambda k, j, idx: (idx[k], j, 0)),
                pl.BlockSpec((1, HBLK, 1), lambda k, j, idx: (idx[k], j, 0)),
                pl.BlockSpec((1, D, HBLK), lambda k, j, idx: (idx[k], 0, j)),
                pl.BlockSpec((1, D, 1), lambda k, j, idx: (idx[k], 0, 0)),
            ],
            out_specs=pl.BlockSpec((D, S), lambda k, j, idx: (0, 0)),
        ),
        out_shape=jax.ShapeDtypeStruct((D, S), jnp.float32),
    )(idx, probs, hnT, xT, fc1, b1, fc2, b2)


def kernel(x_swin, x_conv, noise_eps, params):
    p = params
    x0, xn = _ln2(x_swin.reshape(S, D), x_conv.reshape(S, D),
                  p['swin_pre_ln_g'], p['swin_pre_ln_b'],
                  p['conv_pre_ln_g'], p['conv_pre_ln_b'])
    oc = _stage(xn, p['cross_swin_in_w'], p['cross_swin_in_b'],
                p['cross_conv_in_w'], p['cross_conv_in_b'], cross=True)
    x2, tn = _proj2_ln(oc, p['cross_swin_out_w'], p['cross_swin_out_b'],
                       p['cross_conv_out_w'], p['cross_conv_out_b'],
                       x0,
                       p['swin_self_ln_g'], p['swin_self_ln_b'],
                       p['conv_self_ln_g'], p['conv_self_ln_b'])
    os_ = _stage(tn, p['self_swin_in_w'], p['self_swin_in_b'],
                 p['self_conv_in_w'], p['self_conv_in_b'], cross=False)
    rw = jnp.concatenate([p['router_w'], p['noise_w']], axis=0)
    rb = jnp.concatenate([p['router_b'], p['noise_b']], axis=0)
    xT, hnT, nm = _proj2_moe(os_, p['self_swin_out_w'], p['self_swin_out_b'],
                             p['self_conv_out_w'], p['self_conv_out_b'],
                             x2, p['moe_ln_g'], p['moe_ln_b'], rw, rb,
                             noise_eps.reshape(S, E))

    vals, idx = jax.lax.top_k(nm.reshape(E), K)
    probs = jax.nn.softmax(vals).reshape(1, K)

    outT = _experts(idx.astype(jnp.int32), probs, hnT, xT,
                    p['exp_fc1_w'], p['exp_fc1_b'].reshape(E, HID, 1),
                    p['exp_fc2_w'], p['exp_fc2_b'].reshape(E, D, 1))
    return outT.T.reshape(B, S, D)
